# Initial kernel scaffold; baseline (speedup 1.0000x reference)
import jax
import jax.numpy as jnp
from jax.experimental import pallas as pl


def kernel(x, Wq, bq, Wk, bk):
    raise NotImplementedError("write your pallas kernel here")


# speedup vs baseline: 7.5723x; 7.5723x over previous
import functools

import jax
import jax.numpy as jnp
from jax.experimental import pallas as pl

H = 4
D = 256
ROW_BLOCK = 1000


def _phase_a(x_ref, wkT_ref, bk_ref, kvs_ref, sums_ref):
    j = pl.program_id(0)

    @pl.when(j == 0)
    def _init():
        kvs_ref[...] = jnp.zeros_like(kvs_ref)
        sums_ref[...] = jnp.zeros_like(sums_ref)

    x = x_ref[...]
    rows = []
    for h in range(H):
        k = jnp.dot(x, wkT_ref[:, h * D:(h + 1) * D],
                    preferred_element_type=jnp.float32)
        k = k + bk_ref[0, h * D:(h + 1) * D][None, :]
        k = k * jax.lax.rsqrt(jnp.sum(k * k, axis=1, keepdims=True))
        kvs_ref[h] += jax.lax.dot_general(
            k, x, (((0,), (0,)), ((), ())),
            preferred_element_type=jnp.float32)
        rows.append(jnp.sum(k, axis=0)[None, :])
    rows.append(jnp.sum(x, axis=0)[None, :])
    rows.append(jnp.zeros((3, D), jnp.float32))
    sums_ref[...] += jnp.concatenate(rows, axis=0)


def _phase_b(n_total, x_ref, wqT_ref, bq_ref, kvs_ref, sums_ref, out_ref):
    x = x_ref[...]
    x_sum = sums_ref[H, :]
    for h in range(H):
        q = jnp.dot(x, wqT_ref[:, h * D:(h + 1) * D],
                    preferred_element_type=jnp.float32)
        q = q + bq_ref[0, h * D:(h + 1) * D][None, :]
        q = q * jax.lax.rsqrt(jnp.sum(q * q, axis=1, keepdims=True))
        num = jnp.dot(q, kvs_ref[h], preferred_element_type=jnp.float32)
        num = num + x_sum[None, :]
        den = jnp.sum(q * sums_ref[h, :][None, :], axis=1, keepdims=True)
        den = den + jnp.float32(n_total)
        out_ref[:, h * D:(h + 1) * D] = num / den


def kernel(x, Wq, bq, Wk, bk):
    n, in_ch = x.shape
    assert n % ROW_BLOCK == 0
    nb = n // ROW_BLOCK
    wqT = Wq.T
    wkT = Wk.T
    bq2 = bq[None, :]
    bk2 = bk[None, :]

    kvs, sums = pl.pallas_call(
        _phase_a,
        grid=(nb,),
        in_specs=[
            pl.BlockSpec((ROW_BLOCK, in_ch), lambda j: (j, 0)),
            pl.BlockSpec((in_ch, H * D), lambda j: (0, 0)),
            pl.BlockSpec((1, H * D), lambda j: (0, 0)),
        ],
        out_specs=[
            pl.BlockSpec((H, D, D), lambda j: (0, 0, 0)),
            pl.BlockSpec((8, D), lambda j: (0, 0)),
        ],
        out_shape=[
            jax.ShapeDtypeStruct((H, D, D), jnp.float32),
            jax.ShapeDtypeStruct((8, D), jnp.float32),
        ],
    )(x, wkT, bk2)

    out = pl.pallas_call(
        functools.partial(_phase_b, n),
        grid=(nb,),
        in_specs=[
            pl.BlockSpec((ROW_BLOCK, in_ch), lambda j: (j, 0)),
            pl.BlockSpec((in_ch, H * D), lambda j: (0, 0)),
            pl.BlockSpec((1, H * D), lambda j: (0, 0)),
            pl.BlockSpec((H, D, D), lambda j: (0, 0, 0)),
            pl.BlockSpec((8, D), lambda j: (0, 0)),
        ],
        out_specs=pl.BlockSpec((ROW_BLOCK, H * D), lambda j: (j, 0)),
        out_shape=jax.ShapeDtypeStruct((n, H * D), jnp.float32),
    )(x, wqT, bq2, kvs, sums)
    return out


# speedup vs baseline: 7.8997x; 1.0432x over previous
import functools

import jax
import jax.numpy as jnp
from jax.experimental import pallas as pl

H = 4
D = 256
ROW_BLOCK = 1000


def _phase_a(x_ref, wkT_ref, bk_ref, kvs_ref, sums_ref):
    j = pl.program_id(0)

    @pl.when(j == 0)
    def _init():
        kvs_ref[...] = jnp.zeros_like(kvs_ref)
        sums_ref[...] = jnp.zeros_like(sums_ref)

    x = x_ref[...]
    xb = x.astype(jnp.bfloat16)
    rows = []
    for h in range(H):
        k = jnp.dot(xb, wkT_ref[:, h * D:(h + 1) * D],
                    preferred_element_type=jnp.float32)
        k = k + bk_ref[0, h * D:(h + 1) * D][None, :]
        k = k * jax.lax.rsqrt(jnp.sum(k * k, axis=1, keepdims=True))
        kvs_ref[h] += jax.lax.dot_general(
            k.astype(jnp.bfloat16), xb, (((0,), (0,)), ((), ())),
            preferred_element_type=jnp.float32)
        rows.append(jnp.sum(k, axis=0)[None, :])
    rows.append(jnp.sum(x, axis=0)[None, :])
    rows.append(jnp.zeros((3, D), jnp.float32))
    sums_ref[...] += jnp.concatenate(rows, axis=0)


def _phase_b(n_total, x_ref, wqT_ref, bq_ref, kvs_ref, sums_ref, out_ref):
    xb = x_ref[...].astype(jnp.bfloat16)
    x_sum = sums_ref[H, :]
    for h in range(H):
        q = jnp.dot(xb, wqT_ref[:, h * D:(h + 1) * D],
                    preferred_element_type=jnp.float32)
        q = q + bq_ref[0, h * D:(h + 1) * D][None, :]
        q = q * jax.lax.rsqrt(jnp.sum(q * q, axis=1, keepdims=True))
        num = jnp.dot(q.astype(jnp.bfloat16), kvs_ref[h],
                      preferred_element_type=jnp.float32)
        num = num + x_sum[None, :]
        den = jnp.sum(q * sums_ref[h, :][None, :], axis=1, keepdims=True)
        den = den + jnp.float32(n_total)
        out_ref[:, h * D:(h + 1) * D] = num / den


def kernel(x, Wq, bq, Wk, bk):
    n, in_ch = x.shape
    assert n % ROW_BLOCK == 0
    nb = n // ROW_BLOCK
    wqT = Wq.T.astype(jnp.bfloat16)
    wkT = Wk.T.astype(jnp.bfloat16)
    bq2 = bq[None, :]
    bk2 = bk[None, :]

    kvs, sums = pl.pallas_call(
        _phase_a,
        grid=(nb,),
        in_specs=[
            pl.BlockSpec((ROW_BLOCK, in_ch), lambda j: (j, 0)),
            pl.BlockSpec((in_ch, H * D), lambda j: (0, 0)),
            pl.BlockSpec((1, H * D), lambda j: (0, 0)),
        ],
        out_specs=[
            pl.BlockSpec((H, D, D), lambda j: (0, 0, 0)),
            pl.BlockSpec((8, D), lambda j: (0, 0)),
        ],
        out_shape=[
            jax.ShapeDtypeStruct((H, D, D), jnp.float32),
            jax.ShapeDtypeStruct((8, D), jnp.float32),
        ],
    )(x, wkT, bk2)

    out = pl.pallas_call(
        functools.partial(_phase_b, n),
        grid=(nb,),
        in_specs=[
            pl.BlockSpec((ROW_BLOCK, in_ch), lambda j: (j, 0)),
            pl.BlockSpec((in_ch, H * D), lambda j: (0, 0)),
            pl.BlockSpec((1, H * D), lambda j: (0, 0)),
            pl.BlockSpec((H, D, D), lambda j: (0, 0, 0)),
            pl.BlockSpec((8, D), lambda j: (0, 0)),
        ],
        out_specs=pl.BlockSpec((ROW_BLOCK, H * D), lambda j: (j, 0)),
        out_shape=jax.ShapeDtypeStruct((n, H * D), jnp.float32),
    )(x, wqT, bq2, kvs.astype(jnp.bfloat16), sums)
    return out


# speedup vs baseline: 8.2106x; 1.0394x over previous
import functools

import jax
import jax.numpy as jnp
from jax.experimental import pallas as pl
from jax.experimental.pallas import tpu as pltpu

H = 4
D = 256
ROW_BLOCK = 1000


def _fused(n_total, x_ref, wT_ref, b_ref, out_ref, kvs_ref, sums_ref):
    p = pl.program_id(0)
    j = pl.program_id(1)
    xb = x_ref[...].astype(jnp.bfloat16)

    @pl.when(p == 0)
    def _phase_a():
        @pl.when(j == 0)
        def _init():
            kvs_ref[...] = jnp.zeros_like(kvs_ref)
            sums_ref[...] = jnp.zeros_like(sums_ref)

        rows = []
        for h in range(H):
            k = jnp.dot(xb, wT_ref[0, :, h * D:(h + 1) * D],
                        preferred_element_type=jnp.float32)
            k = k + b_ref[0, 0, h * D:(h + 1) * D][None, :]
            k = k * jax.lax.rsqrt(jnp.sum(k * k, axis=1, keepdims=True))
            kvs_ref[h] += jax.lax.dot_general(
                k.astype(jnp.bfloat16), xb, (((0,), (0,)), ((), ())),
                preferred_element_type=jnp.float32)
            rows.append(jnp.sum(k, axis=0)[None, :])
        rows.append(jnp.sum(x_ref[...], axis=0)[None, :])
        rows.append(jnp.zeros((3, D), jnp.float32))
        sums_ref[...] += jnp.concatenate(rows, axis=0)

    @pl.when(p == 1)
    def _phase_b():
        x_sum = sums_ref[H, :]
        for h in range(H):
            q = jnp.dot(xb, wT_ref[0, :, h * D:(h + 1) * D],
                        preferred_element_type=jnp.float32)
            q = q + b_ref[0, 0, h * D:(h + 1) * D][None, :]
            q = q * jax.lax.rsqrt(jnp.sum(q * q, axis=1, keepdims=True))
            num = jnp.dot(q.astype(jnp.bfloat16),
                          kvs_ref[h].astype(jnp.bfloat16),
                          preferred_element_type=jnp.float32)
            num = num + x_sum[None, :]
            den = jnp.sum(q * sums_ref[h, :][None, :], axis=1, keepdims=True)
            den = den + jnp.float32(n_total)
            out_ref[:, h * D:(h + 1) * D] = num / den


def kernel(x, Wq, bq, Wk, bk):
    n, in_ch = x.shape
    assert n % ROW_BLOCK == 0
    nb = n // ROW_BLOCK
    wT = jnp.stack([Wk.T.astype(jnp.bfloat16), Wq.T.astype(jnp.bfloat16)])
    b2 = jnp.stack([bk[None, :], bq[None, :]])

    out = pl.pallas_call(
        functools.partial(_fused, n),
        grid=(2, nb),
        in_specs=[
            pl.BlockSpec((ROW_BLOCK, in_ch), lambda p, j: (j, 0)),
            pl.BlockSpec((1, in_ch, H * D), lambda p, j: (p, 0, 0)),
            pl.BlockSpec((1, 1, H * D), lambda p, j: (p, 0, 0)),
        ],
        out_specs=pl.BlockSpec((ROW_BLOCK, H * D), lambda p, j: (p * j, 0)),
        out_shape=jax.ShapeDtypeStruct((n, H * D), jnp.float32),
        scratch_shapes=[
            pltpu.VMEM((H, D, D), jnp.float32),
            pltpu.VMEM((8, D), jnp.float32),
        ],
    )(x, wT, b2)
    return out


# speedup vs baseline: 9.2557x; 1.1273x over previous
import functools

import jax
import jax.numpy as jnp
from jax.experimental import pallas as pl
from jax.experimental.pallas import tpu as pltpu

H = 4
D = 256
ROW_BLOCK = 2000


def _fused(n_total, x_ref, wT_ref, b_ref, out_ref, kvs_ref, sums_ref):
    p = pl.program_id(0)
    j = pl.program_id(1)
    xb = x_ref[...].astype(jnp.bfloat16)

    @pl.when(p == 0)
    def _phase_a():
        @pl.when(j == 0)
        def _init():
            kvs_ref[...] = jnp.zeros_like(kvs_ref)
            sums_ref[...] = jnp.zeros_like(sums_ref)

        rows = []
        for h in range(H):
            k = jnp.dot(xb, wT_ref[0, :, h * D:(h + 1) * D],
                        preferred_element_type=jnp.float32)
            k = k + b_ref[0, 0, h * D:(h + 1) * D][None, :]
            k = k * jax.lax.rsqrt(jnp.sum(k * k, axis=1, keepdims=True))
            kvs_ref[h] += jax.lax.dot_general(
                k.astype(jnp.bfloat16), xb, (((0,), (0,)), ((), ())),
                preferred_element_type=jnp.float32)
            rows.append(jnp.sum(k, axis=0)[None, :])
        rows.append(jnp.sum(x_ref[...], axis=0)[None, :])
        rows.append(jnp.zeros((3, D), jnp.float32))
        sums_ref[...] += jnp.concatenate(rows, axis=0)

    @pl.when(p == 1)
    def _phase_b():
        x_sum = sums_ref[H, :]
        for h in range(H):
            q = jnp.dot(xb, wT_ref[0, :, h * D:(h + 1) * D],
                        preferred_element_type=jnp.float32)
            q = q + b_ref[0, 0, h * D:(h + 1) * D][None, :]
            q = q * jax.lax.rsqrt(jnp.sum(q * q, axis=1, keepdims=True))
            num = jnp.dot(q.astype(jnp.bfloat16),
                          kvs_ref[h].astype(jnp.bfloat16),
                          preferred_element_type=jnp.float32)
            num = num + x_sum[None, :]
            den = jnp.sum(q * sums_ref[h, :][None, :], axis=1, keepdims=True)
            den = den + jnp.float32(n_total)
            out_ref[:, h * D:(h + 1) * D] = num / den


def kernel(x, Wq, bq, Wk, bk):
    n, in_ch = x.shape
    assert n % ROW_BLOCK == 0
    nb = n // ROW_BLOCK
    wT = jnp.stack([Wk.T.astype(jnp.bfloat16), Wq.T.astype(jnp.bfloat16)])
    b2 = jnp.stack([bk[None, :], bq[None, :]])

    out = pl.pallas_call(
        functools.partial(_fused, n),
        grid=(2, nb),
        in_specs=[
            pl.BlockSpec((ROW_BLOCK, in_ch), lambda p, j: (j, 0)),
            pl.BlockSpec((1, in_ch, H * D), lambda p, j: (p, 0, 0)),
            pl.BlockSpec((1, 1, H * D), lambda p, j: (p, 0, 0)),
        ],
        out_specs=pl.BlockSpec((ROW_BLOCK, H * D), lambda p, j: (p * j, 0)),
        out_shape=jax.ShapeDtypeStruct((n, H * D), jnp.float32),
        scratch_shapes=[
            pltpu.VMEM((H, D, D), jnp.float32),
            pltpu.VMEM((8, D), jnp.float32),
        ],
    )(x, wT, b2)
    return out


# speedup vs baseline: 9.4680x; 1.0229x over previous
import functools

import jax
import jax.numpy as jnp
from jax.experimental import pallas as pl
from jax.experimental.pallas import tpu as pltpu

H = 4
D = 256
ROW_BLOCK = 5000


def _fused(n_total, x_ref, wT_ref, b_ref, out_ref, kvs_ref, sums_ref):
    p = pl.program_id(0)
    j = pl.program_id(1)
    xb = x_ref[...].astype(jnp.bfloat16)

    @pl.when(p == 0)
    def _phase_a():
        @pl.when(j == 0)
        def _init():
            kvs_ref[...] = jnp.zeros_like(kvs_ref)
            sums_ref[...] = jnp.zeros_like(sums_ref)

        rows = []
        for h in range(H):
            k = jnp.dot(xb, wT_ref[0, :, h * D:(h + 1) * D],
                        preferred_element_type=jnp.float32)
            k = k + b_ref[0, 0, h * D:(h + 1) * D][None, :]
            k = k * jax.lax.rsqrt(jnp.sum(k * k, axis=1, keepdims=True))
            kvs_ref[h] += jax.lax.dot_general(
                k.astype(jnp.bfloat16), xb, (((0,), (0,)), ((), ())),
                preferred_element_type=jnp.float32)
            rows.append(jnp.sum(k, axis=0)[None, :])
        rows.append(jnp.sum(x_ref[...], axis=0)[None, :])
        rows.append(jnp.zeros((3, D), jnp.float32))
        sums_ref[...] += jnp.concatenate(rows, axis=0)

    @pl.when(p == 1)
    def _phase_b():
        x_sum = sums_ref[H, :]
        for h in range(H):
            q = jnp.dot(xb, wT_ref[0, :, h * D:(h + 1) * D],
                        preferred_element_type=jnp.float32)
            q = q + b_ref[0, 0, h * D:(h + 1) * D][None, :]
            q = q * jax.lax.rsqrt(jnp.sum(q * q, axis=1, keepdims=True))
            num = jnp.dot(q.astype(jnp.bfloat16),
                          kvs_ref[h].astype(jnp.bfloat16),
                          preferred_element_type=jnp.float32)
            num = num + x_sum[None, :]
            den = jnp.sum(q * sums_ref[h, :][None, :], axis=1, keepdims=True)
            den = den + jnp.float32(n_total)
            out_ref[:, h * D:(h + 1) * D] = num / den


def kernel(x, Wq, bq, Wk, bk):
    n, in_ch = x.shape
    assert n % ROW_BLOCK == 0
    nb = n // ROW_BLOCK
    wT = jnp.stack([Wk.T.astype(jnp.bfloat16), Wq.T.astype(jnp.bfloat16)])
    b2 = jnp.stack([bk[None, :], bq[None, :]])

    out = pl.pallas_call(
        functools.partial(_fused, n),
        grid=(2, nb),
        in_specs=[
            pl.BlockSpec((ROW_BLOCK, in_ch), lambda p, j: (j, 0)),
            pl.BlockSpec((1, in_ch, H * D), lambda p, j: (p, 0, 0)),
            pl.BlockSpec((1, 1, H * D), lambda p, j: (p, 0, 0)),
        ],
        out_specs=pl.BlockSpec((ROW_BLOCK, H * D), lambda p, j: (p * j, 0)),
        out_shape=jax.ShapeDtypeStruct((n, H * D), jnp.float32),
        scratch_shapes=[
            pltpu.VMEM((H, D, D), jnp.float32),
            pltpu.VMEM((8, D), jnp.float32),
        ],
    )(x, wT, b2)
    return out
